# baseline (speedup 1.0000x reference)
import functools

import jax
import jax.numpy as jnp
from jax import lax
from jax.experimental import pallas as pl
from jax.experimental.pallas import tpu as pltpu
from jax.experimental.pallas import tpu_sc as plsc

D = 100
DPAD = 128
NCODE = 50
CV = 2
IDXW = CV * NCODE
NWORKERS = 32
_OFFS = (0, 16, 32, 48, 64, 80, 84)


def _sc_gather_mean(idx2d, emb, n_visits):
    n_chunks = idx2d.shape[0]
    nit = n_chunks // NWORKERS
    vpw = n_visits // NWORKERS

    mesh = plsc.VectorSubcoreMesh(core_axis_name="c", subcore_axis_name="s")

    nbuf = 2
    nphase = 2

    @functools.partial(
        pl.kernel,
        out_type=jax.ShapeDtypeStruct((n_visits, D), jnp.float32),
        mesh=mesh,
        scratch_types=[
            pltpu.VMEM((nit, IDXW), jnp.int32),
            pltpu.VMEM((nbuf, IDXW, DPAD), jnp.float32),
            pltpu.VMEM((vpw // nphase, D), jnp.float32),
            pltpu.SemaphoreType.DMA,
            pltpu.SemaphoreType.DMA,
        ],
    )
    def run(idx_hbm, emb_hbm, out_hbm, idx_v, rows, out_v, s0, s1):
        wid = lax.axis_index("s") * 2 + lax.axis_index("c")
        sems = (s0, s1)

        pltpu.sync_copy(idx_hbm.at[pl.ds(wid * nit, nit)], idx_v)

        def fire(it, b):
            pltpu.async_copy(emb_hbm.at[idx_v.at[it]], rows.at[b], sems[b])

        def drain(b):
            pltpu.make_async_copy(
                emb_hbm.at[idx_v.at[0]], rows.at[b], sems[b]).wait()

        def reduce_chunk(it, vbase, b):
            for v in range(CV):
                def body(r, acc):
                    for u in range(10):
                        row = v * NCODE + r * 10 + u
                        acc = tuple(
                            acc[j] + rows[b, row, pl.ds(_OFFS[j], 16)]
                            for j in range(len(_OFFS))
                        )
                    return acc
                init = tuple(jnp.zeros((16,), jnp.float32)
                             for _ in range(len(_OFFS)))
                acc = lax.fori_loop(0, NCODE // 10, body, init)
                ovis = it * CV + v - vbase
                for j in range(len(_OFFS)):
                    out_v[ovis, pl.ds(_OFFS[j], 16)] = acc[j]

        hn = nit // nphase
        hv = vpw // nphase
        for h in range(nphase):
            for b in range(nbuf):
                fire(h * hn + b, b)

            def loop(k, _, h=h):
                for b in range(nbuf):
                    it = h * hn + k * nbuf + b
                    drain(b)
                    reduce_chunk(it, h * hv, b)

                    @pl.when(it + nbuf < (h + 1) * hn)
                    def _():
                        fire(it + nbuf, b)
                return 0

            lax.fori_loop(0, hn // nbuf, loop, 0)
            pltpu.sync_copy(out_v, out_hbm.at[pl.ds(wid * vpw + h * hv, hv)])

    return run(idx2d, emb)


def _pad_body(src_ref, dst_ref):
    dst_ref[:, pl.ds(0, D)] = src_ref[...]


def _tc_pad_table(emb):
    v = emb.shape[0]
    blk = 25000
    return pl.pallas_call(
        _pad_body,
        grid=(v // blk,),
        in_specs=[pl.BlockSpec((blk, D), lambda i: (i, 0))],
        out_specs=pl.BlockSpec((blk, DPAD), lambda i: (i, 0)),
        out_shape=jax.ShapeDtypeStruct((v, DPAD), jnp.float32),
    )(emb)


def _mlp_body(nv, ve_ref, w1_ref, b1_ref, w2_ref, b2_ref, out_ref):
    x = ve_ref[...]
    h = jnp.maximum(
        jnp.dot(x, w1_ref[...], preferred_element_type=jnp.float32)
        + b1_ref[...], 0.0)
    tb = out_ref.shape[0]
    hs = h.reshape(tb, nv, h.shape[-1]).sum(axis=1)
    out_ref[...] = (
        jnp.dot(hs, w2_ref[...], preferred_element_type=jnp.float32)
        + nv * b2_ref[...])


def _tc_mlp(visit_emb, W1, b1, W2, b2, batch, nv):
    hdim = W2.shape[-1]
    tb = 256
    grid = batch // tb
    return pl.pallas_call(
        functools.partial(_mlp_body, nv),
        grid=(grid,),
        in_specs=[
            pl.BlockSpec((tb * nv, D), lambda i: (i, 0)),
            pl.BlockSpec((D, hdim), lambda i: (0, 0)),
            pl.BlockSpec((1, hdim), lambda i: (0, 0)),
            pl.BlockSpec((hdim, hdim), lambda i: (0, 0)),
            pl.BlockSpec((1, hdim), lambda i: (0, 0)),
        ],
        out_specs=pl.BlockSpec((tb, hdim), lambda i: (i, 0)),
        out_shape=jax.ShapeDtypeStruct((batch, hdim), jnp.float32),
    )(visit_emb, W1, b1.reshape(1, hdim), W2, b2.reshape(1, hdim))


def kernel(visits, emb, W1, b1, W2, b2):
    batch, nv, nc = visits.shape
    n_visits = batch * nv
    n_chunks = n_visits // CV
    idx2d = visits.reshape(n_chunks, CV * nc).astype(jnp.int32)
    emb_p = _tc_pad_table(emb)
    visit_emb = _sc_gather_mean(idx2d, emb_p, n_visits)
    return _tc_mlp(visit_emb, W1 * jnp.float32(1.0 / nc), b1, W2, b2,
                   batch, nv)


# speedup vs baseline: 1.2820x; 1.2820x over previous
import functools

import jax
import jax.numpy as jnp
from jax import lax
from jax.experimental import pallas as pl
from jax.experimental.pallas import tpu as pltpu
from jax.experimental.pallas import tpu_sc as plsc

D = 100
DPAD = 128
NCODE = 50
CV = 2
IDXW = CV * NCODE
NWORKERS = 32
_OFFS = (0, 16, 32, 48, 64, 80, 84)


def _sc_gather_mean(idx2d, emb, n_visits):
    n_chunks = idx2d.shape[0]
    nit = n_chunks // NWORKERS
    vpw = n_visits // NWORKERS

    mesh = plsc.VectorSubcoreMesh(core_axis_name="c", subcore_axis_name="s")

    nbuf = 4
    nphase = 4

    @functools.partial(
        pl.kernel,
        out_type=jax.ShapeDtypeStruct((n_visits, D), jnp.float32),
        mesh=mesh,
        scratch_types=[
            pltpu.VMEM((nit, IDXW), jnp.int32),
            pltpu.VMEM((nbuf, IDXW, DPAD), jnp.float32),
            pltpu.VMEM((vpw // nphase, D), jnp.float32),
            pltpu.SemaphoreType.DMA,
            pltpu.SemaphoreType.DMA,
            pltpu.SemaphoreType.DMA,
            pltpu.SemaphoreType.DMA,
        ],
    )
    def run(idx_hbm, emb_hbm, out_hbm, idx_v, rows, out_v, s0, s1, s2, s3):
        wid = lax.axis_index("s") * 2 + lax.axis_index("c")
        sems = (s0, s1, s2, s3)

        pltpu.sync_copy(idx_hbm.at[pl.ds(wid * nit, nit)], idx_v)

        def fire(it, b):
            pltpu.async_copy(emb_hbm.at[idx_v.at[it]], rows.at[b], sems[b])

        def drain(b):
            pltpu.make_async_copy(
                emb_hbm.at[idx_v.at[0]], rows.at[b], sems[b]).wait()

        def reduce_chunk(it, vbase, b):
            for v in range(CV):
                def body(r, acc):
                    for u in range(10):
                        row = v * NCODE + r * 10 + u
                        acc = tuple(
                            acc[j] + rows[b, row, pl.ds(_OFFS[j], 16)]
                            for j in range(len(_OFFS))
                        )
                    return acc
                init = tuple(jnp.zeros((16,), jnp.float32)
                             for _ in range(len(_OFFS)))
                acc = lax.fori_loop(0, NCODE // 10, body, init)
                ovis = it * CV + v - vbase
                for j in range(len(_OFFS)):
                    out_v[ovis, pl.ds(_OFFS[j], 16)] = acc[j]

        hn = nit // nphase
        hv = vpw // nphase
        for h in range(nphase):
            for b in range(nbuf):
                fire(h * hn + b, b)

            def loop(k, _, h=h):
                for b in range(nbuf):
                    it = h * hn + k * nbuf + b
                    drain(b)
                    reduce_chunk(it, h * hv, b)

                    @pl.when(it + nbuf < (h + 1) * hn)
                    def _():
                        fire(it + nbuf, b)
                return 0

            lax.fori_loop(0, hn // nbuf, loop, 0)
            pltpu.sync_copy(out_v, out_hbm.at[pl.ds(wid * vpw + h * hv, hv)])

    return run(idx2d, emb)


def _pad_body(src_ref, dst_ref):
    dst_ref[:, pl.ds(0, D)] = src_ref[...]


def _tc_pad_table(emb):
    v = emb.shape[0]
    blk = 25000
    return pl.pallas_call(
        _pad_body,
        grid=(v // blk,),
        in_specs=[pl.BlockSpec((blk, D), lambda i: (i, 0))],
        out_specs=pl.BlockSpec((blk, DPAD), lambda i: (i, 0)),
        out_shape=jax.ShapeDtypeStruct((v, DPAD), jnp.float32),
    )(emb)


def _mlp_body(nv, ve_ref, w1_ref, b1_ref, w2_ref, b2_ref, out_ref):
    x = ve_ref[...]
    h = jnp.maximum(
        jnp.dot(x, w1_ref[...], preferred_element_type=jnp.float32)
        + b1_ref[...], 0.0)
    tb = out_ref.shape[0]
    hs = h.reshape(tb, nv, h.shape[-1]).sum(axis=1)
    out_ref[...] = (
        jnp.dot(hs, w2_ref[...], preferred_element_type=jnp.float32)
        + nv * b2_ref[...])


def _tc_mlp(visit_emb, W1, b1, W2, b2, batch, nv):
    hdim = W2.shape[-1]
    tb = 256
    grid = batch // tb
    return pl.pallas_call(
        functools.partial(_mlp_body, nv),
        grid=(grid,),
        in_specs=[
            pl.BlockSpec((tb * nv, D), lambda i: (i, 0)),
            pl.BlockSpec((D, hdim), lambda i: (0, 0)),
            pl.BlockSpec((1, hdim), lambda i: (0, 0)),
            pl.BlockSpec((hdim, hdim), lambda i: (0, 0)),
            pl.BlockSpec((1, hdim), lambda i: (0, 0)),
        ],
        out_specs=pl.BlockSpec((tb, hdim), lambda i: (i, 0)),
        out_shape=jax.ShapeDtypeStruct((batch, hdim), jnp.float32),
    )(visit_emb, W1, b1.reshape(1, hdim), W2, b2.reshape(1, hdim))


def kernel(visits, emb, W1, b1, W2, b2):
    batch, nv, nc = visits.shape
    n_visits = batch * nv
    n_chunks = n_visits // CV
    idx2d = visits.reshape(n_chunks, CV * nc).astype(jnp.int32)
    emb_p = _tc_pad_table(emb)
    visit_emb = _sc_gather_mean(idx2d, emb_p, n_visits)
    return _tc_mlp(visit_emb, W1 * jnp.float32(1.0 / nc), b1, W2, b2,
                   batch, nv)
